# baseline (speedup 1.0000x reference)
import functools

import jax
import jax.numpy as jnp
from jax.experimental import pallas as pl
from jax.experimental.pallas import tpu as pltpu


def _round_up(x, m):
    return ((x + m - 1) // m) * m


def _fused_sample_kernel_t(ann_ref, epsT_ref, w_ref, zT_ref, *, n_ann):
    ann = ann_ref[0]
    epsT = epsT_ref[...].astype(jnp.bfloat16)
    tbl = epsT.shape[1]

    zero = jnp.bfloat16(0.0)
    n_chunk = 2
    g = n_ann // n_chunk
    kc = g * epsT.shape[0] + g
    zT = None
    for c in range(n_chunk):
        chunks = [jnp.where(ann == a, epsT, zero)
                  for a in range(c * g, (c + 1) * g)]
        iota_a = jax.lax.broadcasted_iota(jnp.int32, (g, tbl), 0) + (c * g)
        onehot = (iota_a == ann).astype(jnp.bfloat16)
        x = jnp.concatenate(chunks + [onehot], axis=0)
        p = jax.lax.dot_general(
            w_ref[c * kc:(c + 1) * kc], x, (((0,), (0,)), ((), ())),
            preferred_element_type=jnp.float32)
        zT = p if zT is None else zT + p
    zT_ref[...] = zT.astype(zT_ref.dtype)


def kernel(posterior_mu, posterior_covtril, annotator, eps):
    posterior_mu = jnp.asarray(posterior_mu, jnp.float32)
    A, D = posterior_mu.shape
    annotator = jnp.asarray(annotator).astype(jnp.int32)
    B = annotator.shape[0]
    eps = jnp.asarray(eps, jnp.float32)

    tile_bl = 8192
    tbl = tile_bl if B >= tile_bl else max(128, _round_up(B, 128))
    b_pad = _round_up(B, tbl)
    epsT = jnp.transpose(eps)
    if b_pad != B:
        annotator = jnp.pad(annotator, (0, b_pad - B))
        epsT = jnp.pad(epsT, ((0, 0), (0, b_pad - B)))
    ann2 = annotator.reshape(b_pad // tbl, 1, tbl)

    l_tril = jnp.tril(jnp.asarray(posterior_covtril, jnp.float32))
    w_l = jnp.transpose(l_tril, (0, 2, 1)).reshape(A * D, D)
    n_chunk = 2
    g = A // n_chunk
    w_parts = []
    for c in range(n_chunk):
        w_parts.append(w_l[c * g * D:(c + 1) * g * D])
        w_parts.append(posterior_mu[c * g:(c + 1) * g])
    w_aug = jnp.concatenate(w_parts, axis=0).astype(jnp.bfloat16)

    grid = (b_pad // tbl,)
    zT = pl.pallas_call(
        functools.partial(_fused_sample_kernel_t, n_ann=A),
        out_shape=jax.ShapeDtypeStruct((D, b_pad), jnp.float32),
        grid=grid,
        in_specs=[
            pl.BlockSpec((1, 1, tbl), lambda i: (i, 0, 0)),
            pl.BlockSpec((D, tbl), lambda i: (0, i)),
            pl.BlockSpec((A * (D + 1), D), lambda i: (0, 0)),
        ],
        out_specs=pl.BlockSpec((D, tbl), lambda i: (0, i)),
        compiler_params=pltpu.CompilerParams(dimension_semantics=("parallel",)),
    )(ann2, epsT, w_aug)
    return jnp.transpose(zT)[:B]


# speedup vs baseline: 1.0433x; 1.0433x over previous
import functools

import jax
import jax.numpy as jnp
from jax.experimental import pallas as pl
from jax.experimental.pallas import tpu as pltpu


def _round_up(x, m):
    return ((x + m - 1) // m) * m


def _fused_sample_kernel_t(ann_ref, epsT_ref, w_ref, zT_ref, *, n_ann):
    ann = ann_ref[0]
    epsT = epsT_ref[...].astype(jnp.bfloat16)
    tbl = epsT.shape[1]

    zero = jnp.bfloat16(0.0)
    chunks = [jnp.where(ann == a, epsT, zero) for a in range(n_ann)]
    iota_a = jax.lax.broadcasted_iota(jnp.int32, (n_ann, tbl), 0)
    onehot = (iota_a == ann).astype(jnp.bfloat16)
    x = jnp.concatenate(chunks + [onehot], axis=0)

    zT = jax.lax.dot_general(
        w_ref[...], x, (((0,), (0,)), ((), ())),
        preferred_element_type=jnp.float32)
    zT_ref[...] = zT.astype(zT_ref.dtype)


def kernel(posterior_mu, posterior_covtril, annotator, eps):
    posterior_mu = jnp.asarray(posterior_mu, jnp.float32)
    A, D = posterior_mu.shape
    annotator = jnp.asarray(annotator).astype(jnp.int32)
    B = annotator.shape[0]
    eps = jnp.asarray(eps, jnp.float32)

    tile_bl = 8192
    tbl = tile_bl if B >= tile_bl else max(128, _round_up(B, 128))
    b_pad = _round_up(B, tbl)
    epsT = jnp.transpose(eps)
    if b_pad != B:
        annotator = jnp.pad(annotator, (0, b_pad - B))
        epsT = jnp.pad(epsT, ((0, 0), (0, b_pad - B)))
    ann2 = annotator.reshape(b_pad // tbl, 1, tbl)

    l_tril = jnp.tril(jnp.asarray(posterior_covtril, jnp.float32))
    w_l = jnp.transpose(l_tril, (0, 2, 1)).reshape(A * D, D)
    w_aug = jnp.concatenate([w_l, posterior_mu], axis=0).astype(jnp.bfloat16)

    grid = (b_pad // tbl,)
    zT = pl.pallas_call(
        functools.partial(_fused_sample_kernel_t, n_ann=A),
        out_shape=jax.ShapeDtypeStruct((D, b_pad), jnp.float32),
        grid=grid,
        in_specs=[
            pl.BlockSpec((1, 1, tbl), lambda i: (i, 0, 0)),
            pl.BlockSpec((D, tbl), lambda i: (0, i)),
            pl.BlockSpec((A * (D + 1), D), lambda i: (0, 0)),
        ],
        out_specs=pl.BlockSpec((D, tbl), lambda i: (0, i)),
        compiler_params=pltpu.CompilerParams(dimension_semantics=("parallel",)),
    )(ann2, epsT, w_aug)
    return jnp.transpose(zT)[:B]
